# baseline (speedup 1.0000x reference)
import functools

import jax
import jax.numpy as jnp
from jax import lax
from jax.experimental import pallas as pl
from jax.experimental.pallas import tpu as pltpu
from jax.experimental.pallas import tpu_sc as plsc

F32 = jnp.float32

N = 10000
E = 320000
HID = 128
G = 128
OUT = 10
NL = 3

NC = 2
NS = 16
NW = NC * NS
EPW = E // NW
CH = 16
NCHUNK = EPW // CH
NPAD = 10240
RPT = NPAD // NS

BR = 1000
NBLK = N // BR

@functools.cache
def _vmesh():
    return plsc.VectorSubcoreMesh(core_axis_name="c", subcore_axis_name="s",
                                  num_cores=NC, num_subcores=NS)


CHD = 80
NCHD = EPW // CHD
NBD = 5


def _sc_degree(dst, ea, zcol):
    @functools.partial(
        pl.kernel,
        out_type=jax.ShapeDtypeStruct((NC, NPAD), F32),
        mesh=_vmesh(),
        scratch_types=[
            pltpu.VMEM_SHARED((NPAD,), F32),
            [pltpu.VMEM((CHD,), jnp.int32)] * NBD,
            [pltpu.VMEM((CHD,), F32)] * NBD,
            [pltpu.SemaphoreType.DMA] * NBD,
            [pltpu.SemaphoreType.DMA] * NBD,
            [pltpu.SemaphoreType.DMA] * NBD,
        ],
        compiler_params=pltpu.CompilerParams(use_tc_tiling_on_sc=False),
    )
    def k(dst_hbm, ea_hbm, z_hbm, dgp_hbm, dg_sh, idx_v, val_v,
          isem, vsem, ssem):
        c = lax.axis_index("c")
        s = lax.axis_index("s")
        base = (c * NS + s) * EPW

        def issue(ch, b):
            off = base + ch * CHD
            pltpu.async_copy(dst_hbm.at[pl.ds(off, CHD)], idx_v[b], isem[b])
            pltpu.async_copy(ea_hbm.at[pl.ds(off, CHD)], val_v[b], vsem[b])

        issue(0, 0)
        issue(1, 1)
        pltpu.sync_copy(z_hbm, dg_sh.at[pl.ds(s * RPT, RPT)])
        plsc.subcore_barrier()

        @pl.loop(0, NCHD, step=NBD)
        def _(i):
            for b in range(NBD):
                ch = i + b
                n2 = (b + 2) % NBD
                off = base + ch * CHD

                @pl.when(ch + 2 < NCHD)
                def _():
                    @pl.when(ch >= NBD - 2)
                    def _():
                        pltpu.make_async_copy(
                            val_v[n2], dg_sh.at[idx_v[n2]], ssem[n2]).wait()
                    issue(ch + 2, n2)

                pltpu.make_async_copy(
                    dst_hbm.at[pl.ds(off, CHD)], idx_v[b], isem[b]).wait()
                pltpu.make_async_copy(
                    ea_hbm.at[pl.ds(off, CHD)], val_v[b], vsem[b]).wait()
                pltpu.async_copy(val_v[b], dg_sh.at[idx_v[b]], ssem[b],
                                 add=True)

        for b in range(NBD):
            pltpu.make_async_copy(val_v[b], dg_sh.at[idx_v[b]], ssem[b]).wait()
        plsc.subcore_barrier()
        pltpu.sync_copy(dg_sh.at[pl.ds(s * RPT, RPT)],
                        dgp_hbm.at[c, pl.ds(s * RPT, RPT)])

    return k(dst, ea, zcol)


NBUF = 5


def _sc_spmm(a, slab, zrows):
    @functools.partial(
        pl.kernel,
        out_type=jax.ShapeDtypeStruct((NC, NPAD, HID), F32),
        mesh=_vmesh(),
        scratch_types=[
            pltpu.VMEM_SHARED((NPAD, HID), F32),
            [pltpu.VMEM((CH, HID), F32)] * NBUF,
            [pltpu.VMEM((2 + CH, 16), jnp.int32)] * NBUF,
            [pltpu.SemaphoreType.DMA] * NBUF,
            [pltpu.SemaphoreType.DMA] * NBUF,
            [pltpu.SemaphoreType.DMA] * NBUF,
        ],
        compiler_params=pltpu.CompilerParams(use_tc_tiling_on_sc=False,
                                             needs_layout_passes=False),
    )
    def k(a_hbm, slab_hbm, z_hbm, p_hbm, acc_sh, rows_v, slab_v,
          gsem, ssem, lsem):
        c = lax.axis_index("c")
        s = lax.axis_index("s")
        cbase = (c * NS + s) * NCHUNK

        def issue_slab(ch, b):
            pltpu.async_copy(slab_hbm.at[cbase + ch], slab_v[b], lsem[b])

        def issue_gather(ch, b):
            pltpu.make_async_copy(
                slab_hbm.at[cbase + ch], slab_v[b], lsem[b]).wait()
            pltpu.async_copy(a_hbm.at[slab_v[b].at[0]], rows_v[b], gsem[b])

        issue_slab(0, 0)
        issue_slab(1, 1)
        pltpu.sync_copy(z_hbm, acc_sh.at[pl.ds(s * RPT, RPT)])
        plsc.subcore_barrier()
        issue_gather(0, 0)

        @pl.loop(0, NCHUNK, step=NBUF)
        def _(i):
            for b in range(NBUF):
                ch = i + b
                n1 = (b + 1) % NBUF
                n2 = (b + 2) % NBUF

                @pl.when(ch + 2 < NCHUNK)
                def _():
                    @pl.when(ch >= NBUF - 2)
                    def _():
                        pltpu.make_async_copy(
                            rows_v[n2], acc_sh.at[slab_v[n2].at[1]],
                            ssem[n2]).wait()
                    issue_slab(ch + 2, n2)

                @pl.when(ch + 1 < NCHUNK)
                def _():
                    issue_gather(ch + 1, n1)

                pltpu.make_async_copy(
                    a_hbm.at[slab_v[b].at[0]], rows_v[b], gsem[b]).wait()

                @pl.loop(0, CH)
                def _(e):
                    wv = plsc.bitcast(slab_v[b][2 + e, :], F32)
                    for j in range(HID // 16):
                        sl = pl.ds(j * 16, 16)
                        rows_v[b][e, sl] = rows_v[b][e, sl] * wv

                pltpu.async_copy(rows_v[b], acc_sh.at[slab_v[b].at[1]],
                                 ssem[b], add=True)

        for b in range(NBUF):
            pltpu.make_async_copy(rows_v[b], acc_sh.at[slab_v[b].at[1]],
                                  ssem[b]).wait()
        plsc.subcore_barrier()
        pltpu.sync_copy(acc_sh.at[pl.ds(s * RPT, RPT)],
                        p_hbm.at[c, pl.ds(s * RPT, RPT)])

    return k(a, slab, zrows)


_FULL_W = pl.BlockSpec((HID, HID), lambda i: (0, 0))
_FULL_B = pl.BlockSpec((1, HID), lambda i: (0, 0))
_ROWS = pl.BlockSpec((BR, HID), lambda i: (i, 0))
_P_BLK = pl.BlockSpec((NC, BR, HID), lambda i: (0, i, 0))
_DG_BLK = pl.BlockSpec((NC, BR, 1), lambda i: (0, i, 0))


def _dense_terms(h, dg, W1, W2, W3, b1, b3, a_out, z_out):
    a_out[...] = jnp.dot(h, W1[...], preferred_element_type=F32)
    z_out[...] = (jnp.dot(h, W3[...], preferred_element_type=F32) + b3[...]
                  + dg * b1[...]
                  - dg * jnp.dot(h, W2[...], preferred_element_type=F32))


def _tc_encoder(x, dgp, encW, encb, W1, W2, W3, b1, b3):
    def body(x_ref, dgp_ref, encW_ref, encb_ref, W1_ref, W2_ref, W3_ref,
             b1_ref, b3_ref, a_out, z_out):
        h = jnp.dot(x_ref[...], encW_ref[...], preferred_element_type=F32) + encb_ref[...]
        dg = dgp_ref[0] + dgp_ref[1]
        _dense_terms(h, dg, W1_ref, W2_ref, W3_ref, b1_ref, b3_ref, a_out, z_out)

    return pl.pallas_call(
        body,
        grid=(NBLK,),
        in_specs=[_ROWS, _DG_BLK, _FULL_W, _FULL_B, _FULL_W, _FULL_W, _FULL_W,
                  _FULL_B, _FULL_B],
        out_specs=[_ROWS, _ROWS],
        out_shape=[jax.ShapeDtypeStruct((N, HID), F32),
                   jax.ShapeDtypeStruct((N, HID), F32)],
    )(x, dgp, encW, encb, W1, W2, W3, b1, b3)


def _tc_layer(p, z, dgp, W1, W2, W3, b1, b3):
    def body(p_ref, z_ref, dgp_ref, W1_ref, W2_ref, W3_ref, b1_ref, b3_ref,
             a_out, z_out):
        h = jnp.maximum(p_ref[0] + p_ref[1] + z_ref[...], 0.0)
        dg = dgp_ref[0] + dgp_ref[1]
        _dense_terms(h, dg, W1_ref, W2_ref, W3_ref, b1_ref, b3_ref, a_out, z_out)

    return pl.pallas_call(
        body,
        grid=(NBLK,),
        in_specs=[_P_BLK, _ROWS, _DG_BLK, _FULL_W, _FULL_W, _FULL_W, _FULL_B,
                  _FULL_B],
        out_specs=[_ROWS, _ROWS],
        out_shape=[jax.ShapeDtypeStruct((N, HID), F32),
                   jax.ShapeDtypeStruct((N, HID), F32)],
    )(p, z, dgp, W1, W2, W3, b1, b3)


def _tc_head(p, z, batch3, fcW, fcb):
    def body(p_ref, z_ref, bat_ref, fcW_ref, fcb_ref, out_ref,
             pooled_ref, cnt_ref):
        i = pl.program_id(0)

        @pl.when(i == 0)
        def _():
            pooled_ref[...] = jnp.zeros((G, HID), F32)
            cnt_ref[...] = jnp.zeros((G, G), F32)

        h = jnp.maximum(p_ref[0] + p_ref[1] + z_ref[...], 0.0)
        bi = bat_ref[0]
        gi = lax.broadcasted_iota(jnp.int32, (G, BR), 0)
        oh = (gi == jnp.broadcast_to(bi, (G, BR))).astype(F32)
        pooled_ref[...] += jnp.dot(oh, h, preferred_element_type=F32)
        cnt_ref[...] += jnp.broadcast_to(
            jnp.sum(oh, axis=1, keepdims=True), (G, G))

        @pl.when(i == NBLK - 1)
        def _():
            gx = pooled_ref[...] / jnp.maximum(cnt_ref[...], 1.0)
            out_ref[...] = (jnp.dot(gx, fcW_ref[...], preferred_element_type=F32)
                            + fcb_ref[...])

    return pl.pallas_call(
        body,
        grid=(NBLK,),
        in_specs=[_P_BLK, _ROWS,
                  pl.BlockSpec((1, 1, BR), lambda i: (i, 0, 0)),
                  pl.BlockSpec((HID, OUT), lambda i: (0, 0)),
                  pl.BlockSpec((1, OUT), lambda i: (0, 0))],
        out_specs=pl.BlockSpec((G, OUT), lambda i: (0, 0)),
        out_shape=jax.ShapeDtypeStruct((G, OUT), F32),
        scratch_shapes=[pltpu.VMEM((G, HID), F32), pltpu.VMEM((G, G), F32)],
    )(p, z, batch3, fcW, fcb)


def kernel(x, edge_index, edge_attr, batch, enc_W, enc_b, lin1_W, lin1_b,
           lin2_W, lin3_W, lin3_b, fc_W, fc_b):
    src = edge_index[0]
    dst = edge_index[1]
    nck = E // CH
    wbits = jax.lax.bitcast_convert_type(edge_attr, jnp.int32)
    slab = jnp.concatenate([
        src.reshape(nck, 1, CH),
        dst.reshape(nck, 1, CH),
        jnp.broadcast_to(wbits.reshape(nck, CH)[:, :, None], (nck, CH, 16)),
    ], axis=1)
    zcol = jnp.zeros((RPT,), F32)
    zrows = jnp.zeros((RPT, HID), F32)
    batch3 = batch.reshape(NBLK, 1, BR)
    encb = enc_b[None, :]
    fcb = fc_b[None, :]

    dgp = _sc_degree(dst, edge_attr, zcol)[:, :, None]

    a, z = _tc_encoder(x, dgp, enc_W, encb,
                       lin1_W[0], lin2_W[0], lin3_W[0],
                       lin1_b[0][None, :], lin3_b[0][None, :])
    for l in range(1, NL):
        p = _sc_spmm(a, slab, zrows)
        a, z = _tc_layer(p, z, dgp,
                         lin1_W[l], lin2_W[l], lin3_W[l],
                         lin1_b[l][None, :], lin3_b[l][None, :])
    p = _sc_spmm(a, slab, zrows)
    return _tc_head(p, z, batch3, fc_W, fcb)


# speedup vs baseline: 1.1513x; 1.1513x over previous
import functools

import jax
import jax.numpy as jnp
from jax import lax
from jax.experimental import pallas as pl
from jax.experimental.pallas import tpu as pltpu
from jax.experimental.pallas import tpu_sc as plsc

F32 = jnp.float32

N = 10000
E = 320000
HID = 128
G = 128
OUT = 10
NL = 3

NC = 2
NS = 16
NW = NC * NS
EPW = E // NW
CH = 16
NCHUNK = EPW // CH
NPAD = 10240
RPT = NPAD // NS

BR = 1000
NBLK = N // BR

@functools.cache
def _vmesh():
    return plsc.VectorSubcoreMesh(core_axis_name="c", subcore_axis_name="s",
                                  num_cores=NC, num_subcores=NS)


CHD = 80
NCHD = EPW // CHD
NBD = 5


def _sc_degree(dst, ea, zcol):
    @functools.partial(
        pl.kernel,
        out_type=jax.ShapeDtypeStruct((NC, NPAD), F32),
        mesh=_vmesh(),
        scratch_types=[
            pltpu.VMEM_SHARED((NPAD,), F32),
            [pltpu.VMEM((CHD,), jnp.int32)] * NBD,
            [pltpu.VMEM((CHD,), F32)] * NBD,
            [pltpu.SemaphoreType.DMA] * NBD,
            [pltpu.SemaphoreType.DMA] * NBD,
            [pltpu.SemaphoreType.DMA] * NBD,
        ],
        compiler_params=pltpu.CompilerParams(use_tc_tiling_on_sc=False),
    )
    def k(dst_hbm, ea_hbm, z_hbm, dgp_hbm, dg_sh, idx_v, val_v,
          isem, vsem, ssem):
        c = lax.axis_index("c")
        s = lax.axis_index("s")
        base = (c * NS + s) * EPW

        def issue(ch, b):
            off = base + ch * CHD
            pltpu.async_copy(dst_hbm.at[pl.ds(off, CHD)], idx_v[b], isem[b])
            pltpu.async_copy(ea_hbm.at[pl.ds(off, CHD)], val_v[b], vsem[b])

        issue(0, 0)
        issue(1, 1)
        pltpu.sync_copy(z_hbm, dg_sh.at[pl.ds(s * RPT, RPT)])
        plsc.subcore_barrier()

        @pl.loop(0, NCHD, step=NBD)
        def _(i):
            for b in range(NBD):
                ch = i + b
                n2 = (b + 2) % NBD
                off = base + ch * CHD

                @pl.when(ch + 2 < NCHD)
                def _():
                    @pl.when(ch >= NBD - 2)
                    def _():
                        pltpu.make_async_copy(
                            val_v[n2], dg_sh.at[idx_v[n2]], ssem[n2]).wait()
                    issue(ch + 2, n2)

                pltpu.make_async_copy(
                    dst_hbm.at[pl.ds(off, CHD)], idx_v[b], isem[b]).wait()
                pltpu.make_async_copy(
                    ea_hbm.at[pl.ds(off, CHD)], val_v[b], vsem[b]).wait()
                pltpu.async_copy(val_v[b], dg_sh.at[idx_v[b]], ssem[b],
                                 add=True)

        for b in range(NBD):
            pltpu.make_async_copy(val_v[b], dg_sh.at[idx_v[b]], ssem[b]).wait()
        plsc.subcore_barrier()
        pltpu.sync_copy(dg_sh.at[pl.ds(s * RPT, RPT)],
                        dgp_hbm.at[c, pl.ds(s * RPT, RPT)])

    return k(dst, ea, zcol)


NBUF = 5


SLW = CH * 18


def _sc_spmm(a, slab, zrows):
    @functools.partial(
        pl.kernel,
        out_type=jax.ShapeDtypeStruct((NC, NPAD, HID), F32),
        mesh=_vmesh(),
        scratch_types=[
            pltpu.VMEM_SHARED((NPAD, HID), F32),
            [pltpu.VMEM((CH, HID), F32)] * NBUF,
            [pltpu.VMEM((SLW,), jnp.int32)] * NBUF,
            [pltpu.VMEM((CH,), jnp.int32)] * NBUF,
            [pltpu.VMEM((CH,), jnp.int32)] * NBUF,
            [pltpu.SemaphoreType.DMA] * NBUF,
            [pltpu.SemaphoreType.DMA] * NBUF,
            [pltpu.SemaphoreType.DMA] * NBUF,
        ],
        compiler_params=pltpu.CompilerParams(needs_layout_passes=False),
    )
    def k(a_hbm, slab_hbm, z_hbm, p_hbm, acc_sh, rows_v, slab_v, sidx_v,
          didx_v, gsem, ssem, lsem):
        c = lax.axis_index("c")
        s = lax.axis_index("s")
        cbase = (c * NS + s) * NCHUNK

        def issue_slab(ch, b):
            pltpu.async_copy(slab_hbm.at[pl.ds((cbase + ch) * SLW, SLW)],
                             slab_v[b], lsem[b])

        def issue_gather(ch, b):
            pltpu.make_async_copy(
                slab_hbm.at[pl.ds((cbase + ch) * SLW, SLW)], slab_v[b],
                lsem[b]).wait()
            for kk in range(CH // 16):
                sl = pl.ds(kk * 16, 16)
                sidx_v[b][sl] = slab_v[b][sl]
            pltpu.async_copy(a_hbm.at[sidx_v[b]], rows_v[b], gsem[b])

        issue_slab(0, 0)
        issue_slab(1, 1)
        pltpu.sync_copy(z_hbm, acc_sh.at[pl.ds(s * RPT, RPT)])
        plsc.subcore_barrier()
        issue_gather(0, 0)

        @pl.loop(0, NCHUNK, step=NBUF)
        def _(i):
            for b in range(NBUF):
                ch = i + b
                n1 = (b + 1) % NBUF
                n2 = (b + 2) % NBUF

                @pl.when(ch + 2 < NCHUNK)
                def _():
                    @pl.when(ch >= NBUF - 2)
                    def _():
                        pltpu.make_async_copy(
                            rows_v[n2], acc_sh.at[didx_v[n2]],
                            ssem[n2]).wait()
                    issue_slab(ch + 2, n2)

                @pl.when(ch + 1 < NCHUNK)
                def _():
                    issue_gather(ch + 1, n1)

                for kk in range(CH // 16):
                    didx_v[b][pl.ds(kk * 16, 16)] = slab_v[b][
                        pl.ds(CH + kk * 16, 16)]

                pltpu.make_async_copy(
                    a_hbm.at[sidx_v[b]], rows_v[b], gsem[b]).wait()

                @pl.loop(0, CH)
                def _(e):
                    wv = plsc.bitcast(slab_v[b][pl.ds(2 * CH + e * 16, 16)],
                                      F32)
                    for j in range(HID // 16):
                        sl = pl.ds(j * 16, 16)
                        rows_v[b][e, sl] = rows_v[b][e, sl] * wv

                pltpu.async_copy(rows_v[b], acc_sh.at[didx_v[b]],
                                 ssem[b], add=True)

        for b in range(NBUF):
            pltpu.make_async_copy(rows_v[b], acc_sh.at[didx_v[b]],
                                  ssem[b]).wait()
        plsc.subcore_barrier()
        pltpu.sync_copy(acc_sh.at[pl.ds(s * RPT, RPT)],
                        p_hbm.at[c, pl.ds(s * RPT, RPT)])

    return k(a, slab, zrows)


_FULL_W = pl.BlockSpec((HID, HID), lambda i: (0, 0))
_FULL_B = pl.BlockSpec((1, HID), lambda i: (0, 0))
_ROWS = pl.BlockSpec((BR, HID), lambda i: (i, 0))
_P_BLK = pl.BlockSpec((NC, BR, HID), lambda i: (0, i, 0))
_DG_BLK = pl.BlockSpec((NC, BR, 1), lambda i: (0, i, 0))


def _dense_terms(h, dg, W1, W2, W3, b1, b3, a_out, z_out):
    a_out[...] = jnp.dot(h, W1[...], preferred_element_type=F32)
    z_out[...] = (jnp.dot(h, W3[...], preferred_element_type=F32) + b3[...]
                  + dg * b1[...]
                  - dg * jnp.dot(h, W2[...], preferred_element_type=F32))


def _tc_encoder(x, dgp, encW, encb, W1, W2, W3, b1, b3):
    def body(x_ref, dgp_ref, encW_ref, encb_ref, W1_ref, W2_ref, W3_ref,
             b1_ref, b3_ref, a_out, z_out):
        h = jnp.dot(x_ref[...], encW_ref[...], preferred_element_type=F32) + encb_ref[...]
        dg = dgp_ref[0] + dgp_ref[1]
        _dense_terms(h, dg, W1_ref, W2_ref, W3_ref, b1_ref, b3_ref, a_out, z_out)

    return pl.pallas_call(
        body,
        grid=(NBLK,),
        in_specs=[_ROWS, _DG_BLK, _FULL_W, _FULL_B, _FULL_W, _FULL_W, _FULL_W,
                  _FULL_B, _FULL_B],
        out_specs=[_ROWS, _ROWS],
        out_shape=[jax.ShapeDtypeStruct((N, HID), F32),
                   jax.ShapeDtypeStruct((N, HID), F32)],
    )(x, dgp, encW, encb, W1, W2, W3, b1, b3)


def _tc_layer(p, z, dgp, W1, W2, W3, b1, b3):
    def body(p_ref, z_ref, dgp_ref, W1_ref, W2_ref, W3_ref, b1_ref, b3_ref,
             a_out, z_out):
        h = jnp.maximum(p_ref[0] + p_ref[1] + z_ref[...], 0.0)
        dg = dgp_ref[0] + dgp_ref[1]
        _dense_terms(h, dg, W1_ref, W2_ref, W3_ref, b1_ref, b3_ref, a_out, z_out)

    return pl.pallas_call(
        body,
        grid=(NBLK,),
        in_specs=[_P_BLK, _ROWS, _DG_BLK, _FULL_W, _FULL_W, _FULL_W, _FULL_B,
                  _FULL_B],
        out_specs=[_ROWS, _ROWS],
        out_shape=[jax.ShapeDtypeStruct((N, HID), F32),
                   jax.ShapeDtypeStruct((N, HID), F32)],
    )(p, z, dgp, W1, W2, W3, b1, b3)


def _tc_head(p, z, batch3, fcW, fcb):
    def body(p_ref, z_ref, bat_ref, fcW_ref, fcb_ref, out_ref,
             pooled_ref, cnt_ref):
        i = pl.program_id(0)

        @pl.when(i == 0)
        def _():
            pooled_ref[...] = jnp.zeros((G, HID), F32)
            cnt_ref[...] = jnp.zeros((G, G), F32)

        h = jnp.maximum(p_ref[0] + p_ref[1] + z_ref[...], 0.0)
        bi = bat_ref[0]
        gi = lax.broadcasted_iota(jnp.int32, (G, BR), 0)
        oh = (gi == jnp.broadcast_to(bi, (G, BR))).astype(F32)
        pooled_ref[...] += jnp.dot(oh, h, preferred_element_type=F32)
        cnt_ref[...] += jnp.broadcast_to(
            jnp.sum(oh, axis=1, keepdims=True), (G, G))

        @pl.when(i == NBLK - 1)
        def _():
            gx = pooled_ref[...] / jnp.maximum(cnt_ref[...], 1.0)
            out_ref[...] = (jnp.dot(gx, fcW_ref[...], preferred_element_type=F32)
                            + fcb_ref[...])

    return pl.pallas_call(
        body,
        grid=(NBLK,),
        in_specs=[_P_BLK, _ROWS,
                  pl.BlockSpec((1, 1, BR), lambda i: (i, 0, 0)),
                  pl.BlockSpec((HID, OUT), lambda i: (0, 0)),
                  pl.BlockSpec((1, OUT), lambda i: (0, 0))],
        out_specs=pl.BlockSpec((G, OUT), lambda i: (0, 0)),
        out_shape=jax.ShapeDtypeStruct((G, OUT), F32),
        scratch_shapes=[pltpu.VMEM((G, HID), F32), pltpu.VMEM((G, G), F32)],
    )(p, z, batch3, fcW, fcb)


def kernel(x, edge_index, edge_attr, batch, enc_W, enc_b, lin1_W, lin1_b,
           lin2_W, lin3_W, lin3_b, fc_W, fc_b):
    src = edge_index[0]
    dst = edge_index[1]
    nck = E // CH
    wbits = jax.lax.bitcast_convert_type(edge_attr, jnp.int32)
    slab = jnp.concatenate([
        src.reshape(nck, CH),
        dst.reshape(nck, CH),
        jnp.broadcast_to(wbits.reshape(nck, CH)[:, :, None],
                         (nck, CH, 16)).reshape(nck, CH * 16),
    ], axis=1).reshape(nck * SLW)
    zcol = jnp.zeros((RPT,), F32)
    zrows = jnp.zeros((RPT, HID), F32)
    batch3 = batch.reshape(NBLK, 1, BR)
    encb = enc_b[None, :]
    fcb = fc_b[None, :]

    dgp = _sc_degree(dst, edge_attr, zcol)[:, :, None]

    a, z = _tc_encoder(x, dgp, enc_W, encb,
                       lin1_W[0], lin2_W[0], lin3_W[0],
                       lin1_b[0][None, :], lin3_b[0][None, :])
    for l in range(1, NL):
        p = _sc_spmm(a, slab, zrows)
        a, z = _tc_layer(p, z, dgp,
                         lin1_W[l], lin2_W[l], lin3_W[l],
                         lin1_b[l][None, :], lin3_b[l][None, :])
    p = _sc_spmm(a, slab, zrows)
    return _tc_head(p, z, batch3, fc_W, fcb)


# speedup vs baseline: 1.7680x; 1.5357x over previous
import functools

import jax
import jax.numpy as jnp
from jax import lax
from jax.experimental import pallas as pl
from jax.experimental.pallas import tpu as pltpu
from jax.experimental.pallas import tpu_sc as plsc

F32 = jnp.float32

N = 10000
E = 320000
HID = 128
G = 128
OUT = 10
NL = 3

NC = 2
NS = 16
NW = NC * NS
EPW = E // NW
EPP = 10240
EP = EPP * NW
CH = 64
NCHUNK = EPP // CH
NPAD = 10240
RPT = NPAD // NS

BR = 1000
NBLK = N // BR

@functools.cache
def _vmesh():
    return plsc.VectorSubcoreMesh(core_axis_name="c", subcore_axis_name="s",
                                  num_cores=NC, num_subcores=NS)


CHD = 80
NCHD = EPW // CHD
NBD = 5


def _sc_degree(dst, ea, zcol):
    @functools.partial(
        pl.kernel,
        out_type=jax.ShapeDtypeStruct((NC, NPAD), F32),
        mesh=_vmesh(),
        scratch_types=[
            pltpu.VMEM_SHARED((NPAD,), F32),
            [pltpu.VMEM((CHD,), jnp.int32)] * NBD,
            [pltpu.VMEM((CHD,), F32)] * NBD,
            [pltpu.SemaphoreType.DMA] * NBD,
            [pltpu.SemaphoreType.DMA] * NBD,
            [pltpu.SemaphoreType.DMA] * NBD,
        ],
        compiler_params=pltpu.CompilerParams(use_tc_tiling_on_sc=False),
    )
    def k(dst_hbm, ea_hbm, z_hbm, dgp_hbm, dg_sh, idx_v, val_v,
          isem, vsem, ssem):
        c = lax.axis_index("c")
        s = lax.axis_index("s")
        base = (c * NS + s) * EPW

        def issue(ch, b):
            off = base + ch * CHD
            pltpu.async_copy(dst_hbm.at[pl.ds(off, CHD)], idx_v[b], isem[b])
            pltpu.async_copy(ea_hbm.at[pl.ds(off, CHD)], val_v[b], vsem[b])

        issue(0, 0)
        issue(1, 1)
        pltpu.sync_copy(z_hbm, dg_sh.at[pl.ds(s * RPT, RPT)])
        plsc.subcore_barrier()

        @pl.loop(0, NCHD, step=NBD)
        def _(i):
            for b in range(NBD):
                ch = i + b
                n2 = (b + 2) % NBD
                off = base + ch * CHD

                @pl.when(ch + 2 < NCHD)
                def _():
                    @pl.when(ch >= NBD - 2)
                    def _():
                        pltpu.make_async_copy(
                            val_v[n2], dg_sh.at[idx_v[n2]], ssem[n2]).wait()
                    issue(ch + 2, n2)

                pltpu.make_async_copy(
                    dst_hbm.at[pl.ds(off, CHD)], idx_v[b], isem[b]).wait()
                pltpu.make_async_copy(
                    ea_hbm.at[pl.ds(off, CHD)], val_v[b], vsem[b]).wait()
                pltpu.async_copy(val_v[b], dg_sh.at[idx_v[b]], ssem[b],
                                 add=True)

        for b in range(NBD):
            pltpu.make_async_copy(val_v[b], dg_sh.at[idx_v[b]], ssem[b]).wait()
        plsc.subcore_barrier()
        pltpu.sync_copy(dg_sh.at[pl.ds(s * RPT, RPT)],
                        dgp_hbm.at[c, pl.ds(s * RPT, RPT)])

    return k(dst, ea, zcol)


NBUF = 4
EU = 8


def _sc_spmm(a, src, dst, wflat, zrows):
    @functools.partial(
        pl.kernel,
        out_type=jax.ShapeDtypeStruct((NC, NPAD, HID), F32),
        mesh=_vmesh(),
        scratch_types=[
            pltpu.VMEM_SHARED((NPAD, HID), F32),
            [pltpu.VMEM((CH, HID), F32)] * NBUF,
            [pltpu.VMEM((CH,), jnp.int32)] * NBUF,
            [pltpu.VMEM((CH,), jnp.int32)] * NBUF,
            [pltpu.VMEM((CH * 16,), F32)] * NBUF,
            [pltpu.SemaphoreType.DMA] * NBUF,
            [pltpu.SemaphoreType.DMA] * NBUF,
            [pltpu.SemaphoreType.DMA] * NBUF,
            [pltpu.SemaphoreType.DMA] * NBUF,
            [pltpu.SemaphoreType.DMA] * NBUF,
        ],
    )
    def k(a_hbm, src_hbm, dst_hbm, w_hbm, z_hbm, p_hbm,
          acc_sh, rows_v, sidx_v, didx_v, w_v, gsem, ssem, xsem, dsem, wsem):
        c = lax.axis_index("c")
        s = lax.axis_index("s")
        base = (c * NS + s) * EPP

        def issue_idx(ch, b):
            off = base + ch * CH
            pltpu.async_copy(src_hbm.at[pl.ds(off, CH)], sidx_v[b], xsem[b])
            pltpu.async_copy(dst_hbm.at[pl.ds(off, CH)], didx_v[b], dsem[b])
            pltpu.async_copy(w_hbm.at[pl.ds(off * 16, CH * 16)], w_v[b],
                             wsem[b])

        def issue_gather(ch, b):
            off = base + ch * CH
            pltpu.make_async_copy(
                src_hbm.at[pl.ds(off, CH)], sidx_v[b], xsem[b]).wait()
            pltpu.async_copy(a_hbm.at[sidx_v[b]], rows_v[b], gsem[b])

        issue_idx(0, 0)
        issue_idx(1, 1)
        pltpu.sync_copy(z_hbm, acc_sh.at[pl.ds(s * RPT, RPT)])
        plsc.subcore_barrier()
        issue_gather(0, 0)

        @pl.loop(0, NCHUNK, step=NBUF)
        def _(i):
            for b in range(NBUF):
                ch = i + b
                n1 = (b + 1) % NBUF
                n2 = (b + 2) % NBUF

                @pl.when(ch + 2 < NCHUNK)
                def _():
                    @pl.when(ch >= NBUF - 2)
                    def _():
                        pltpu.make_async_copy(
                            rows_v[n2], acc_sh.at[didx_v[n2]],
                            ssem[n2]).wait()
                    issue_idx(ch + 2, n2)

                @pl.when(ch + 1 < NCHUNK)
                def _():
                    issue_gather(ch + 1, n1)

                off = base + ch * CH
                pltpu.make_async_copy(
                    a_hbm.at[sidx_v[b]], rows_v[b], gsem[b]).wait()
                pltpu.make_async_copy(
                    w_hbm.at[pl.ds(off * 16, CH * 16)], w_v[b], wsem[b]).wait()

                @pl.loop(0, CH, step=EU)
                def _(e0):
                    for de in range(EU):
                        e = e0 + de
                        wv = w_v[b][pl.ds(e * 16, 16)]
                        for j in range(HID // 16):
                            sl = pl.ds(j * 16, 16)
                            rows_v[b][e, sl] = rows_v[b][e, sl] * wv

                pltpu.make_async_copy(
                    dst_hbm.at[pl.ds(off, CH)], didx_v[b], dsem[b]).wait()
                pltpu.async_copy(rows_v[b], acc_sh.at[didx_v[b]],
                                 ssem[b], add=True)

        for b in range(NBUF):
            pltpu.make_async_copy(rows_v[b], acc_sh.at[didx_v[b]],
                                  ssem[b]).wait()
        plsc.subcore_barrier()
        pltpu.sync_copy(acc_sh.at[pl.ds(s * RPT, RPT)],
                        p_hbm.at[c, pl.ds(s * RPT, RPT)])

    return k(a, src, dst, wflat, zrows)


_FULL_W = pl.BlockSpec((HID, HID), lambda i: (0, 0))
_FULL_B = pl.BlockSpec((1, HID), lambda i: (0, 0))
_ROWS = pl.BlockSpec((BR, HID), lambda i: (i, 0))
_P_BLK = pl.BlockSpec((NC, BR, HID), lambda i: (0, i, 0))
_DG_BLK = pl.BlockSpec((NC, BR, 1), lambda i: (0, i, 0))


def _dense_terms(h, dg, W1, W2, W3, b1, b3, a_out, z_out):
    a_out[...] = jnp.dot(h, W1[...], preferred_element_type=F32)
    z_out[...] = (jnp.dot(h, W3[...], preferred_element_type=F32) + b3[...]
                  + dg * b1[...]
                  - dg * jnp.dot(h, W2[...], preferred_element_type=F32))


def _tc_encoder(x, dgp, encW, encb, W1, W2, W3, b1, b3):
    def body(x_ref, dgp_ref, encW_ref, encb_ref, W1_ref, W2_ref, W3_ref,
             b1_ref, b3_ref, a_out, z_out):
        h = jnp.dot(x_ref[...], encW_ref[...], preferred_element_type=F32) + encb_ref[...]
        dg = dgp_ref[0] + dgp_ref[1]
        _dense_terms(h, dg, W1_ref, W2_ref, W3_ref, b1_ref, b3_ref, a_out, z_out)

    return pl.pallas_call(
        body,
        grid=(NBLK,),
        in_specs=[_ROWS, _DG_BLK, _FULL_W, _FULL_B, _FULL_W, _FULL_W, _FULL_W,
                  _FULL_B, _FULL_B],
        out_specs=[_ROWS, _ROWS],
        out_shape=[jax.ShapeDtypeStruct((N, HID), F32),
                   jax.ShapeDtypeStruct((N, HID), F32)],
    )(x, dgp, encW, encb, W1, W2, W3, b1, b3)


def _tc_layer(p, z, dgp, W1, W2, W3, b1, b3):
    def body(p_ref, z_ref, dgp_ref, W1_ref, W2_ref, W3_ref, b1_ref, b3_ref,
             a_out, z_out):
        h = jnp.maximum(p_ref[0] + p_ref[1] + z_ref[...], 0.0)
        dg = dgp_ref[0] + dgp_ref[1]
        _dense_terms(h, dg, W1_ref, W2_ref, W3_ref, b1_ref, b3_ref, a_out, z_out)

    return pl.pallas_call(
        body,
        grid=(NBLK,),
        in_specs=[_P_BLK, _ROWS, _DG_BLK, _FULL_W, _FULL_W, _FULL_W, _FULL_B,
                  _FULL_B],
        out_specs=[_ROWS, _ROWS],
        out_shape=[jax.ShapeDtypeStruct((N, HID), F32),
                   jax.ShapeDtypeStruct((N, HID), F32)],
    )(p, z, dgp, W1, W2, W3, b1, b3)


def _tc_head(p, z, batch3, fcW, fcb):
    def body(p_ref, z_ref, bat_ref, fcW_ref, fcb_ref, out_ref,
             pooled_ref, cnt_ref):
        i = pl.program_id(0)

        @pl.when(i == 0)
        def _():
            pooled_ref[...] = jnp.zeros((G, HID), F32)
            cnt_ref[...] = jnp.zeros((G, G), F32)

        h = jnp.maximum(p_ref[0] + p_ref[1] + z_ref[...], 0.0)
        bi = bat_ref[0]
        gi = lax.broadcasted_iota(jnp.int32, (G, BR), 0)
        oh = (gi == jnp.broadcast_to(bi, (G, BR))).astype(F32)
        pooled_ref[...] += jnp.dot(oh, h, preferred_element_type=F32)
        cnt_ref[...] += jnp.broadcast_to(
            jnp.sum(oh, axis=1, keepdims=True), (G, G))

        @pl.when(i == NBLK - 1)
        def _():
            gx = pooled_ref[...] / jnp.maximum(cnt_ref[...], 1.0)
            out_ref[...] = (jnp.dot(gx, fcW_ref[...], preferred_element_type=F32)
                            + fcb_ref[...])

    return pl.pallas_call(
        body,
        grid=(NBLK,),
        in_specs=[_P_BLK, _ROWS,
                  pl.BlockSpec((1, 1, BR), lambda i: (i, 0, 0)),
                  pl.BlockSpec((HID, OUT), lambda i: (0, 0)),
                  pl.BlockSpec((1, OUT), lambda i: (0, 0))],
        out_specs=pl.BlockSpec((G, OUT), lambda i: (0, 0)),
        out_shape=jax.ShapeDtypeStruct((G, OUT), F32),
        scratch_shapes=[pltpu.VMEM((G, HID), F32), pltpu.VMEM((G, G), F32)],
    )(p, z, batch3, fcW, fcb)


def kernel(x, edge_index, edge_attr, batch, enc_W, enc_b, lin1_W, lin1_b,
           lin2_W, lin3_W, lin3_b, fc_W, fc_b):
    src = edge_index[0]
    dst = edge_index[1]
    pad = EP - E
    pidx = jnp.arange(pad, dtype=jnp.int32)
    src_p = jnp.concatenate([src, pidx % N])
    dst_p = jnp.concatenate([dst, N + pidx % (NPAD - N)])
    w_p = jnp.concatenate([edge_attr, jnp.zeros((pad,), F32)])
    wflat = jnp.broadcast_to(w_p[:, None], (EP, 16)).reshape(EP * 16)
    zcol = jnp.zeros((RPT,), F32)
    zrows = jnp.zeros((RPT, HID), F32)
    batch3 = batch.reshape(NBLK, 1, BR)
    encb = enc_b[None, :]
    fcb = fc_b[None, :]

    dgp = _sc_degree(dst, edge_attr, zcol)[:, :, None]

    a, z = _tc_encoder(x, dgp, enc_W, encb,
                       lin1_W[0], lin2_W[0], lin3_W[0],
                       lin1_b[0][None, :], lin3_b[0][None, :])
    for l in range(1, NL):
        p = _sc_spmm(a, src_p, dst_p, wflat, zrows)
        a, z = _tc_layer(p, z, dgp,
                         lin1_W[l], lin2_W[l], lin3_W[l],
                         lin1_b[l][None, :], lin3_b[l][None, :])
    p = _sc_spmm(a, src_p, dst_p, wflat, zrows)
    return _tc_head(p, z, batch3, fc_W, fcb)
